# baseline (speedup 1.0000x reference)
import functools

import jax
import jax.numpy as jnp
from jax import lax
from jax.experimental import pallas as pl
from jax.experimental.pallas import tpu as pltpu
from jax.experimental.pallas import tpu_sc as plsc

N = 10000
E = 320000
D = 128
DH = D // 2
NC = 2
NS = 16
NW = NC * NS
K = 125
EPW = E // NW
NCHUNK = EPW // K
EPT = E // NS
NCHUNK2 = EPT // K
RPT = 624
TAIL0 = N - NS * RPT
NBUF = 8
HNCH = NCHUNK2 // 2
WCH = 104
NWCH = RPT // WCH
NG = DH // 16

_MESH = plsc.VectorSubcoreMesh(core_axis_name="c", subcore_axis_name="s")


def _sc_degree(dst3, zeros_n, ones_k):
    @functools.partial(
        pl.kernel,
        out_type=jax.ShapeDtypeStruct((NC * N,), jnp.float32),
        mesh=_MESH,
        scratch_types=[
            pltpu.VMEM((NCHUNK, K), jnp.int32),
            pltpu.VMEM((K,), jnp.float32),
            pltpu.VMEM((RPT,), jnp.float32),
            pltpu.VMEM_SHARED((N,), jnp.float32),
        ],
    )
    def deg_kernel(dst_hbm, zeros_hbm, ones_hbm, out_hbm, dst_v, ones_v,
                   zbuf, acc_sh):
        c = lax.axis_index("c")
        s = lax.axis_index("s")
        wid = c * NS + s
        pltpu.sync_copy(zeros_hbm, zbuf)
        pltpu.sync_copy(zbuf, acc_sh.at[pl.ds(s * RPT, RPT)])

        @pl.when(s == 0)
        def _():
            pltpu.sync_copy(zbuf.at[pl.ds(0, TAIL0)],
                            acc_sh.at[pl.ds(NS * RPT, TAIL0)])

        pltpu.sync_copy(dst_hbm.at[wid], dst_v)
        pltpu.sync_copy(ones_hbm, ones_v)
        plsc.subcore_barrier()

        @pl.loop(0, NCHUNK)
        def _(j):
            pltpu.sync_copy(ones_v, acc_sh.at[dst_v.at[j]], add=True)

        plsc.subcore_barrier()
        pltpu.sync_copy(acc_sh.at[pl.ds(s * RPT, RPT)], zbuf)
        pltpu.sync_copy(zbuf, out_hbm.at[pl.ds(c * N + s * RPT, RPT)])

        @pl.when(s == 0)
        def _():
            pltpu.sync_copy(acc_sh.at[pl.ds(NS * RPT, TAIL0)],
                            ones_v.at[pl.ds(0, TAIL0)])
            pltpu.sync_copy(ones_v.at[pl.ds(0, TAIL0)],
                            out_hbm.at[pl.ds(c * N + NS * RPT, TAIL0)])

    return deg_kernel(dst3, zeros_n, ones_k)


def _sc_layer(yL, yR, src3, dst3, zeros_k, dinvB, bias2, relu):
    @functools.partial(
        pl.kernel,
        out_type=jax.ShapeDtypeStruct((N, D), jnp.float32),
        mesh=_MESH,
        scratch_types=[
            pltpu.VMEM((HNCH, K), jnp.int32),
            pltpu.VMEM((HNCH, K), jnp.int32),
            [pltpu.VMEM((K, DH), jnp.float32)] * NBUF,
            pltpu.VMEM((DH,), jnp.float32),
            pltpu.VMEM_SHARED((N, DH), jnp.float32),
            [pltpu.SemaphoreType.DMA] * NBUF,
            [pltpu.SemaphoreType.DMA] * NBUF,
        ],
        compiler_params=pltpu.CompilerParams(use_tc_tiling_on_sc=False),
    )
    def layer_kernel(yl_hbm, yr_hbm, src_hbm, dst_hbm, zeros_hbm, dinvb_hbm,
                     bias_hbm, out_hbm, src_v, dst_v, rows, bbuf, acc_sh,
                     gsems, ssems):
        c = lax.axis_index("c")
        s = lax.axis_index("s")
        pltpu.sync_copy(zeros_hbm, rows[0])

        @pl.loop(0, NWCH)
        def _(i):
            pltpu.sync_copy(rows[0].at[pl.ds(0, WCH)],
                            acc_sh.at[pl.ds(s * RPT + i * WCH, WCH)])

        @pl.when(s == 0)
        def _():
            pltpu.sync_copy(rows[0].at[pl.ds(0, TAIL0)],
                            acc_sh.at[pl.ds(NS * RPT, TAIL0)])

        plsc.subcore_barrier()

        def run(y_hbm):
            for h in range(2):
                pltpu.sync_copy(src_hbm.at[s, pl.ds(h * HNCH, HNCH)], src_v)
                pltpu.sync_copy(dst_hbm.at[s, pl.ds(h * HNCH, HNCH)], dst_v)
                for b in range(NBUF):
                    pltpu.async_copy(y_hbm.at[src_v.at[b]], rows[b],
                                     gsems[b])

                @pl.loop(0, HNCH, step=NBUF)
                def _(j):
                    sdescs = []
                    for b in range(NBUF):
                        pltpu.make_async_copy(y_hbm.at[src_v.at[j + b]],
                                              rows[b], gsems[b]).wait()
                        sdescs.append(pltpu.async_copy(
                            rows[b], acc_sh.at[dst_v.at[j + b]], ssems[b],
                            add=True))
                    for b in range(NBUF):
                        sdescs[b].wait()

                        @pl.when(j + b + NBUF < HNCH)
                        def _():
                            pltpu.async_copy(
                                y_hbm.at[src_v.at[j + b + NBUF]],
                                rows[b], gsems[b])

        @pl.when(c == 0)
        def _():
            run(yl_hbm)

        @pl.when(c == 1)
        def _():
            run(yr_hbm)

        plsc.subcore_barrier()

        pltpu.sync_copy(bias_hbm.at[pl.ds(c * DH, DH)], bbuf)
        bv = [bbuf[pl.ds(g * 16, 16)] for g in range(NG)]

        def epi_chunk(y_hbm, row0, nr):
            pltpu.sync_copy(acc_sh.at[pl.ds(row0, nr)],
                            rows[0].at[pl.ds(0, nr)])
            pltpu.sync_copy(y_hbm.at[pl.ds(row0, nr)],
                            rows[1].at[pl.ds(0, nr)])
            pltpu.sync_copy(dinvb_hbm.at[pl.ds(row0, nr)],
                            rows[2].at[pl.ds(0, nr)])

            @pl.loop(0, nr)
            def _(r):
                for g in range(NG):
                    sl = pl.ds(g * 16, 16)
                    v = (rows[0][r, sl] + rows[1][r, sl]) * rows[2][r, sl]
                    v = v + bv[g]
                    if relu:
                        v = jnp.maximum(v, 0.0)
                    rows[0][r, sl] = v

            pltpu.sync_copy(
                rows[0].at[pl.ds(0, nr)],
                out_hbm.at[pl.ds(row0, nr), pl.ds(c * DH, DH)])

        def epi_all(y_hbm):
            @pl.loop(0, NWCH)
            def _(i):
                epi_chunk(y_hbm, s * RPT + i * WCH, WCH)

            @pl.when(s == 0)
            def _():
                epi_chunk(y_hbm, NS * RPT, TAIL0)

        @pl.when(c == 0)
        def _():
            epi_all(yl_hbm)

        @pl.when(c == 1)
        def _():
            epi_all(yr_hbm)

    return layer_kernel(yL, yR, src3, dst3, zeros_k, dinvB, bias2)


BR = 2000


def _tc_pre(x, W1, d0, d1):
    def body(x_ref, w_ref, d0_ref, d1_ref, yl_ref, yr_ref, dinvb_ref):
        dinv = lax.rsqrt(d0_ref[...] + d1_ref[...] + 1.0)
        xw = jnp.dot(x_ref[...], w_ref[...],
                     preferred_element_type=jnp.float32) * dinv
        yl_ref[...] = xw[:, :DH]
        yr_ref[...] = xw[:, DH:]
        dinvb_ref[...] = jnp.broadcast_to(dinv, (BR, DH))

    return pl.pallas_call(
        body,
        grid=(N // BR,),
        in_specs=[
            pl.BlockSpec((BR, D), lambda i: (i, 0)),
            pl.BlockSpec((D, D), lambda i: (0, 0)),
            pl.BlockSpec((BR, 1), lambda i: (i, 0)),
            pl.BlockSpec((BR, 1), lambda i: (i, 0)),
        ],
        out_specs=[
            pl.BlockSpec((BR, DH), lambda i: (i, 0)),
            pl.BlockSpec((BR, DH), lambda i: (i, 0)),
            pl.BlockSpec((BR, DH), lambda i: (i, 0)),
        ],
        out_shape=[
            jax.ShapeDtypeStruct((N, DH), jnp.float32),
            jax.ShapeDtypeStruct((N, DH), jnp.float32),
            jax.ShapeDtypeStruct((N, DH), jnp.float32),
        ],
    )(x, W1, d0, d1)


def _tc_mid(h, dinvB, W2):
    def body(h_ref, dinvb_ref, w_ref, y2l_ref, y2r_ref):
        dinv = dinvb_ref[...][:, :1]
        y2 = jnp.dot(h_ref[...], w_ref[...],
                     preferred_element_type=jnp.float32) * dinv
        y2l_ref[...] = y2[:, :DH]
        y2r_ref[...] = y2[:, DH:]

    return pl.pallas_call(
        body,
        grid=(N // BR,),
        in_specs=[
            pl.BlockSpec((BR, D), lambda i: (i, 0)),
            pl.BlockSpec((BR, DH), lambda i: (i, 0)),
            pl.BlockSpec((D, D), lambda i: (0, 0)),
        ],
        out_specs=[
            pl.BlockSpec((BR, DH), lambda i: (i, 0)),
            pl.BlockSpec((BR, DH), lambda i: (i, 0)),
        ],
        out_shape=[
            jax.ShapeDtypeStruct((N, DH), jnp.float32),
            jax.ShapeDtypeStruct((N, DH), jnp.float32),
        ],
    )(h, dinvB, W2)


def kernel(x, edge_index, W1, b1, W2, b2):
    src = edge_index[0].astype(jnp.int32)
    dst = edge_index[1].astype(jnp.int32)
    src3 = src.reshape(NS, NCHUNK2, K)
    dst3 = dst.reshape(NS, NCHUNK2, K)
    dst3d = dst.reshape(NW, NCHUNK, K)
    zeros_n = jnp.zeros((RPT,), jnp.float32)
    zeros_k = jnp.zeros((K, DH), jnp.float32)
    ones_k = jnp.ones((K,), jnp.float32)

    degp = _sc_degree(dst3d, zeros_n, ones_k).reshape(NC, N)
    d0 = degp[0].reshape(N, 1)
    d1 = degp[1].reshape(N, 1)

    y1L, y1R, dinvB = _tc_pre(x, W1, d0, d1)
    h = _sc_layer(y1L, y1R, src3, dst3, zeros_k, dinvB, b1, relu=True)
    y2L, y2R = _tc_mid(h, dinvB, W2)
    return _sc_layer(y2L, y2R, src3, dst3, zeros_k, dinvB, b2, relu=False)


# speedup vs baseline: 1.0647x; 1.0647x over previous
import functools

import jax
import jax.numpy as jnp
from jax import lax
from jax.experimental import pallas as pl
from jax.experimental.pallas import tpu as pltpu
from jax.experimental.pallas import tpu_sc as plsc

N = 10000
E = 320000
D = 128
DH = D // 2
NC = 2
NS = 16
NW = NC * NS
K = 125
EPW = E // NW
NCHUNK = EPW // K
EPT = E // NS
NCHUNK2 = EPT // K
RPT = 624
TAIL0 = N - NS * RPT
NBUF = 8
HNCH = NCHUNK2 // 2
WCH = 104
NWCH = RPT // WCH
NG = DH // 16

_MESH = plsc.VectorSubcoreMesh(core_axis_name="c", subcore_axis_name="s")


def _sc_degree(dst3, zeros_n, ones_k):
    @functools.partial(
        pl.kernel,
        out_type=jax.ShapeDtypeStruct((NC * N,), jnp.float32),
        mesh=_MESH,
        scratch_types=[
            pltpu.VMEM((NCHUNK, K), jnp.int32),
            pltpu.VMEM((K,), jnp.float32),
            pltpu.VMEM((RPT,), jnp.float32),
            pltpu.VMEM_SHARED((N,), jnp.float32),
        ],
    )
    def deg_kernel(dst_hbm, zeros_hbm, ones_hbm, out_hbm, dst_v, ones_v,
                   zbuf, acc_sh):
        c = lax.axis_index("c")
        s = lax.axis_index("s")
        wid = c * NS + s
        pltpu.sync_copy(zeros_hbm, zbuf)
        pltpu.sync_copy(zbuf, acc_sh.at[pl.ds(s * RPT, RPT)])

        @pl.when(s == 0)
        def _():
            pltpu.sync_copy(zbuf.at[pl.ds(0, TAIL0)],
                            acc_sh.at[pl.ds(NS * RPT, TAIL0)])

        pltpu.sync_copy(dst_hbm.at[wid], dst_v)
        pltpu.sync_copy(ones_hbm, ones_v)
        plsc.subcore_barrier()

        @pl.loop(0, NCHUNK)
        def _(j):
            pltpu.sync_copy(ones_v, acc_sh.at[dst_v.at[j]], add=True)

        plsc.subcore_barrier()
        pltpu.sync_copy(acc_sh.at[pl.ds(s * RPT, RPT)], zbuf)
        pltpu.sync_copy(zbuf, out_hbm.at[pl.ds(c * N + s * RPT, RPT)])

        @pl.when(s == 0)
        def _():
            pltpu.sync_copy(acc_sh.at[pl.ds(NS * RPT, TAIL0)],
                            ones_v.at[pl.ds(0, TAIL0)])
            pltpu.sync_copy(ones_v.at[pl.ds(0, TAIL0)],
                            out_hbm.at[pl.ds(c * N + NS * RPT, TAIL0)])

    return deg_kernel(dst3, zeros_n, ones_k)


def _sc_layer(yL, yR, src3, dst3, dinvB, bias2, relu):
    @functools.partial(
        pl.kernel,
        out_type=jax.ShapeDtypeStruct((N, D), jnp.float32),
        mesh=_MESH,
        scratch_types=[
            pltpu.VMEM((HNCH, K), jnp.int32),
            pltpu.VMEM((HNCH, K), jnp.int32),
            [pltpu.VMEM((K, DH), jnp.float32)] * NBUF,
            pltpu.VMEM((DH,), jnp.float32),
            pltpu.VMEM_SHARED((N, DH), jnp.float32),
            [pltpu.SemaphoreType.DMA] * NBUF,
            [pltpu.SemaphoreType.DMA] * NBUF,
        ],
        compiler_params=pltpu.CompilerParams(use_tc_tiling_on_sc=False),
    )
    def layer_kernel(yl_hbm, yr_hbm, src_hbm, dst_hbm, dinvb_hbm,
                     bias_hbm, out_hbm, src_v, dst_v, rows, bbuf, acc_sh,
                     gsems, ssems):
        c = lax.axis_index("c")
        s = lax.axis_index("s")

        def init_acc(y_hbm):
            pltpu.async_copy(y_hbm.at[pl.ds(s * RPT, WCH)],
                             rows[0].at[pl.ds(0, WCH)], gsems[0])
            for i in range(NWCH):
                b = i % 2
                nxt = (i + 1) % 2
                pltpu.make_async_copy(
                    y_hbm.at[pl.ds(s * RPT + i * WCH, WCH)],
                    rows[b].at[pl.ds(0, WCH)], gsems[b]).wait()
                if i + 1 < NWCH:
                    pltpu.async_copy(
                        y_hbm.at[pl.ds(s * RPT + (i + 1) * WCH, WCH)],
                        rows[nxt].at[pl.ds(0, WCH)], gsems[nxt])
                pltpu.sync_copy(rows[b].at[pl.ds(0, WCH)],
                                acc_sh.at[pl.ds(s * RPT + i * WCH, WCH)])

            @pl.when(s == 0)
            def _():
                pltpu.sync_copy(y_hbm.at[pl.ds(NS * RPT, TAIL0)],
                                rows[0].at[pl.ds(0, TAIL0)])
                pltpu.sync_copy(rows[0].at[pl.ds(0, TAIL0)],
                                acc_sh.at[pl.ds(NS * RPT, TAIL0)])

        @pl.when(c == 0)
        def _():
            init_acc(yl_hbm)

        @pl.when(c == 1)
        def _():
            init_acc(yr_hbm)

        plsc.subcore_barrier()

        def run(y_hbm):
            for h in range(2):
                pltpu.sync_copy(src_hbm.at[s, pl.ds(h * HNCH, HNCH)], src_v)
                pltpu.sync_copy(dst_hbm.at[s, pl.ds(h * HNCH, HNCH)], dst_v)
                for b in range(NBUF):
                    pltpu.async_copy(y_hbm.at[src_v.at[b]], rows[b],
                                     gsems[b])

                @pl.loop(0, HNCH, step=NBUF)
                def _(j):
                    sdescs = []
                    for b in range(NBUF):
                        pltpu.make_async_copy(y_hbm.at[src_v.at[j + b]],
                                              rows[b], gsems[b]).wait()
                        sdescs.append(pltpu.async_copy(
                            rows[b], acc_sh.at[dst_v.at[j + b]], ssems[b],
                            add=True))
                    for b in range(NBUF):
                        sdescs[b].wait()

                        @pl.when(j + b + NBUF < HNCH)
                        def _():
                            pltpu.async_copy(
                                y_hbm.at[src_v.at[j + b + NBUF]],
                                rows[b], gsems[b])

        @pl.when(c == 0)
        def _():
            run(yl_hbm)

        @pl.when(c == 1)
        def _():
            run(yr_hbm)

        plsc.subcore_barrier()

        pltpu.sync_copy(bias_hbm.at[pl.ds(c * DH, DH)], bbuf)
        bv = [bbuf[pl.ds(g * 16, 16)] for g in range(NG)]

        def fire_loads(i, nr):
            a, d = rows[2 * (i % 2)], rows[2 * (i % 2) + 1]
            row0 = s * RPT + i * WCH
            pltpu.async_copy(acc_sh.at[pl.ds(row0, nr)],
                             a.at[pl.ds(0, nr)], gsems[i % 2])
            pltpu.async_copy(dinvb_hbm.at[pl.ds(row0, nr)],
                             d.at[pl.ds(0, nr)], gsems[2 + i % 2])

        def wait_loads(i, nr):
            a, d = rows[2 * (i % 2)], rows[2 * (i % 2) + 1]
            row0 = s * RPT + i * WCH
            pltpu.make_async_copy(acc_sh.at[pl.ds(row0, nr)],
                                  a.at[pl.ds(0, nr)], gsems[i % 2]).wait()
            pltpu.make_async_copy(dinvb_hbm.at[pl.ds(row0, nr)],
                                  d.at[pl.ds(0, nr)], gsems[2 + i % 2]).wait()

        def compute_store(i, row0, nr):
            a, d = rows[2 * (i % 2)], rows[2 * (i % 2) + 1]

            @pl.loop(0, nr)
            def _(r):
                for g in range(NG):
                    sl = pl.ds(g * 16, 16)
                    v = a[r, sl] * d[r, sl] + bv[g]
                    if relu:
                        v = jnp.maximum(v, 0.0)
                    a[r, sl] = v

            return pltpu.async_copy(
                a.at[pl.ds(0, nr)],
                out_hbm.at[pl.ds(row0, nr), pl.ds(c * DH, DH)],
                ssems[i % 2])

        fire_loads(0, WCH)
        sdescs = [None, None]
        for i in range(NWCH):
            wait_loads(i, WCH)
            if i + 1 < NWCH:
                fire_loads(i + 1, WCH)
            if sdescs[i % 2] is not None:
                sdescs[i % 2].wait()
            sdescs[i % 2] = compute_store(i, s * RPT + i * WCH, WCH)
        for sd in sdescs:
            sd.wait()

        @pl.when(s == 0)
        def _():
            row0 = NS * RPT
            pltpu.sync_copy(acc_sh.at[pl.ds(row0, TAIL0)],
                            rows[0].at[pl.ds(0, TAIL0)])
            pltpu.sync_copy(dinvb_hbm.at[pl.ds(row0, TAIL0)],
                            rows[1].at[pl.ds(0, TAIL0)])

            @pl.loop(0, TAIL0)
            def _(r):
                for g in range(NG):
                    sl = pl.ds(g * 16, 16)
                    v = rows[0][r, sl] * rows[1][r, sl] + bv[g]
                    if relu:
                        v = jnp.maximum(v, 0.0)
                    rows[0][r, sl] = v

            pltpu.sync_copy(
                rows[0].at[pl.ds(0, TAIL0)],
                out_hbm.at[pl.ds(row0, TAIL0), pl.ds(c * DH, DH)])

    return layer_kernel(yL, yR, src3, dst3, dinvB, bias2)


BR = 2000


def _tc_pre(x, W1, d0, d1):
    def body(x_ref, w_ref, d0_ref, d1_ref, yl_ref, yr_ref, dinvb_ref):
        dinv = lax.rsqrt(d0_ref[...] + d1_ref[...] + 1.0)
        xw = jnp.dot(x_ref[...], w_ref[...],
                     preferred_element_type=jnp.float32) * dinv
        yl_ref[...] = xw[:, :DH]
        yr_ref[...] = xw[:, DH:]
        dinvb_ref[...] = jnp.broadcast_to(dinv, (BR, DH))

    return pl.pallas_call(
        body,
        grid=(N // BR,),
        in_specs=[
            pl.BlockSpec((BR, D), lambda i: (i, 0)),
            pl.BlockSpec((D, D), lambda i: (0, 0)),
            pl.BlockSpec((BR, 1), lambda i: (i, 0)),
            pl.BlockSpec((BR, 1), lambda i: (i, 0)),
        ],
        out_specs=[
            pl.BlockSpec((BR, DH), lambda i: (i, 0)),
            pl.BlockSpec((BR, DH), lambda i: (i, 0)),
            pl.BlockSpec((BR, DH), lambda i: (i, 0)),
        ],
        out_shape=[
            jax.ShapeDtypeStruct((N, DH), jnp.float32),
            jax.ShapeDtypeStruct((N, DH), jnp.float32),
            jax.ShapeDtypeStruct((N, DH), jnp.float32),
        ],
    )(x, W1, d0, d1)


def _tc_mid(h, dinvB, W2):
    def body(h_ref, dinvb_ref, w_ref, y2l_ref, y2r_ref):
        dinv = dinvb_ref[...][:, :1]
        y2 = jnp.dot(h_ref[...], w_ref[...],
                     preferred_element_type=jnp.float32) * dinv
        y2l_ref[...] = y2[:, :DH]
        y2r_ref[...] = y2[:, DH:]

    return pl.pallas_call(
        body,
        grid=(N // BR,),
        in_specs=[
            pl.BlockSpec((BR, D), lambda i: (i, 0)),
            pl.BlockSpec((BR, DH), lambda i: (i, 0)),
            pl.BlockSpec((D, D), lambda i: (0, 0)),
        ],
        out_specs=[
            pl.BlockSpec((BR, DH), lambda i: (i, 0)),
            pl.BlockSpec((BR, DH), lambda i: (i, 0)),
        ],
        out_shape=[
            jax.ShapeDtypeStruct((N, DH), jnp.float32),
            jax.ShapeDtypeStruct((N, DH), jnp.float32),
        ],
    )(h, dinvB, W2)


def kernel(x, edge_index, W1, b1, W2, b2):
    src = edge_index[0].astype(jnp.int32)
    dst = edge_index[1].astype(jnp.int32)
    src3 = src.reshape(NS, NCHUNK2, K)
    dst3 = dst.reshape(NS, NCHUNK2, K)
    dst3d = dst.reshape(NW, NCHUNK, K)
    zeros_n = jnp.zeros((RPT,), jnp.float32)
    ones_k = jnp.ones((K,), jnp.float32)

    degp = _sc_degree(dst3d, zeros_n, ones_k).reshape(NC, N)
    d0 = degp[0].reshape(N, 1)
    d1 = degp[1].reshape(N, 1)

    y1L, y1R, dinvB = _tc_pre(x, W1, d0, d1)
    h = _sc_layer(y1L, y1R, src3, dst3, dinvB, b1, relu=True)
    y2L, y2R = _tc_mid(h, dinvB, W2)
    return _sc_layer(y2L, y2R, src3, dst3, dinvB, b2, relu=False)
